# Initial kernel scaffold; baseline (speedup 1.0000x reference)
import jax
import jax.numpy as jnp
from jax.experimental import pallas as pl


def kernel(X, edge_index, W, b):
    raise NotImplementedError("write your pallas kernel here")


# speedup vs baseline: 22.6995x; 22.6995x over previous
import functools

import jax
import jax.numpy as jnp
from jax import lax
from jax.experimental import pallas as pl
from jax.experimental.pallas import tpu as pltpu
from jax.experimental.pallas import tpu_sc as plsc

N = 10000
E = 320000
D = 128
C = 40

NC = 2
NS = 16
NW = NC * NS
CHUNK = 128
CP = 48
K = -(-E // (NW * CHUNK))
E_PAD = NW * CHUNK * K
N_ACC = ((N + 1 + NW * 8 - 1) // (NW * 8)) * (NW * 8)
RPS = N_ACC // NS

_mesh = plsc.VectorSubcoreMesh(
    core_axis_name="c", subcore_axis_name="s", num_cores=NC, num_subcores=NS
)
_sc_params = pltpu.CompilerParams(use_tc_tiling_on_sc=False)


@functools.partial(
    pl.kernel,
    out_type=jax.ShapeDtypeStruct((NC, N_ACC, 16), jnp.float32),
    mesh=_mesh,
    scratch_types=[
        pltpu.VMEM((K, CHUNK), jnp.int32),
        pltpu.VMEM((CHUNK, 16), jnp.float32),
        pltpu.VMEM_SHARED((N_ACC, 16), jnp.float32),
    ],
    compiler_params=_sc_params,
)
def _deg_kernel(rows_hbm, zeros16_hbm, ones16_hbm, out_hbm, rowv, onesv, acc):
    c = lax.axis_index("c")
    s = lax.axis_index("s")
    wid = c * NS + s
    pltpu.sync_copy(rows_hbm.at[wid], rowv)
    pltpu.sync_copy(ones16_hbm, onesv)
    pltpu.sync_copy(zeros16_hbm.at[pl.ds(s * RPS, RPS)], acc.at[pl.ds(s * RPS, RPS)])
    plsc.subcore_barrier()

    @pl.loop(0, K)
    def _(k):
        pltpu.sync_copy(onesv, acc.at[rowv.at[k]], add=True)

    plsc.subcore_barrier()
    pltpu.sync_copy(acc.at[pl.ds(s * RPS, RPS)], out_hbm.at[c, pl.ds(s * RPS, RPS)])


@functools.partial(
    pl.kernel,
    out_type=jax.ShapeDtypeStruct((NC, N_ACC, CP), jnp.float32),
    mesh=_mesh,
    scratch_types=[
        pltpu.VMEM((K, CHUNK), jnp.int32),
        pltpu.VMEM((K, CHUNK), jnp.int32),
        pltpu.VMEM((CHUNK, CP), jnp.float32),
        pltpu.VMEM_SHARED((N_ACC, CP), jnp.float32),
    ],
    compiler_params=_sc_params,
)
def _prop_kernel(z_hbm, zeros48_hbm, rows_hbm, cols_hbm, out_hbm,
                 rowv, colv, gbuf, acc):
    c = lax.axis_index("c")
    s = lax.axis_index("s")
    wid = c * NS + s
    pltpu.sync_copy(rows_hbm.at[wid], rowv)
    pltpu.sync_copy(cols_hbm.at[wid], colv)

    @pl.when(c == 0)
    def _():
        pltpu.sync_copy(z_hbm.at[pl.ds(s * RPS, RPS)], acc.at[pl.ds(s * RPS, RPS)])

    @pl.when(c == 1)
    def _():
        pltpu.sync_copy(zeros48_hbm.at[pl.ds(s * RPS, RPS)],
                        acc.at[pl.ds(s * RPS, RPS)])

    plsc.subcore_barrier()

    @pl.loop(0, K)
    def _(k):
        pltpu.sync_copy(z_hbm.at[colv.at[k]], gbuf)
        pltpu.sync_copy(gbuf, acc.at[rowv.at[k]], add=True)

    plsc.subcore_barrier()
    pltpu.sync_copy(acc.at[pl.ds(s * RPS, RPS)], out_hbm.at[c, pl.ds(s * RPS, RPS)])


def _mm_body(x_ref, w_ref, cnt_ref, z_ref, deg_ref):
    cnt = cnt_ref[...]
    deg = 1.0 + cnt[0, :, 0] + cnt[1, :, 0]
    y = jnp.dot(x_ref[...], w_ref[...], preferred_element_type=jnp.float32)
    z_ref[...] = y * lax.rsqrt(deg)[:, None]
    deg_ref[...] = deg[:, None]


def _tc_matmul(xp, wp, cnt):
    return pl.pallas_call(
        _mm_body,
        out_shape=(
            jax.ShapeDtypeStruct((N_ACC, CP), jnp.float32),
            jax.ShapeDtypeStruct((N_ACC, 1), jnp.float32),
        ),
    )(xp, wp, cnt)


def _mid_body(t_ref, deg_ref, z_ref):
    t = t_ref[...]
    z_ref[...] = (t[0] + t[1]) / deg_ref[...]


def _tc_mid(t, deg):
    return pl.pallas_call(
        _mid_body,
        out_shape=jax.ShapeDtypeStruct((N_ACC, CP), jnp.float32),
    )(t, deg)


def _fin_body(t_ref, deg_ref, b_ref, o_ref):
    t = t_ref[...]
    o_ref[...] = (t[0] + t[1]) * lax.rsqrt(deg_ref[...]) + b_ref[...]


def _tc_fin(t, deg, bp):
    return pl.pallas_call(
        _fin_body,
        out_shape=jax.ShapeDtypeStruct((N_ACC, CP), jnp.float32),
    )(t, deg, bp)


@jax.jit
def kernel(X, edge_index, W, b):
    ei = edge_index.astype(jnp.int32)
    pad = E_PAD - E
    rows = jnp.concatenate([ei[:, 0], jnp.full((pad,), N, jnp.int32)])
    cols = jnp.concatenate([ei[:, 1], jnp.zeros((pad,), jnp.int32)])
    rows = rows.reshape(NW, K, CHUNK)
    cols = cols.reshape(NW, K, CHUNK)

    xp = jnp.zeros((N_ACC, D), jnp.float32).at[:N].set(X)
    wp = jnp.zeros((D, CP), jnp.float32).at[:, :C].set(W.T)
    bp = jnp.zeros((1, CP), jnp.float32).at[0, :C].set(b)

    zeros16 = jnp.zeros((N_ACC, 16), jnp.float32)
    zeros48 = jnp.zeros((N_ACC, CP), jnp.float32)
    ones16 = jnp.ones((CHUNK, 16), jnp.float32)

    cnt = _deg_kernel(rows, zeros16, ones16)
    z0, deg = _tc_matmul(xp, wp, cnt)
    t1 = _prop_kernel(z0, zeros48, rows, cols)
    z1 = _tc_mid(t1, deg)
    t2 = _prop_kernel(z1, zeros48, rows, cols)
    out = _tc_fin(t2, deg, bp)
    return out[:N, :C]


# speedup vs baseline: 44.5794x; 1.9639x over previous
import functools

import jax
import jax.numpy as jnp
from jax import lax
from jax.experimental import pallas as pl
from jax.experimental.pallas import tpu as pltpu
from jax.experimental.pallas import tpu_sc as plsc

N = 10000
E = 320000
D = 128
C = 40

NC = 2
NS = 16
NW = NC * NS
CHUNK = 128
CP = 48
NBUF = 4
K = ((-(-E // (NW * CHUNK)) + NBUF - 1) // NBUF) * NBUF
E_PAD = NW * CHUNK * K
N_ACC = ((N + CHUNK + NW * 8 - 1) // (NW * 8)) * (NW * 8)
RPS = N_ACC // NS

_mesh = plsc.VectorSubcoreMesh(
    core_axis_name="c", subcore_axis_name="s", num_cores=NC, num_subcores=NS
)
_sc_params = pltpu.CompilerParams(use_tc_tiling_on_sc=False)


@functools.partial(
    pl.kernel,
    out_type=jax.ShapeDtypeStruct((NC, N_ACC, 16), jnp.float32),
    mesh=_mesh,
    scratch_types=[
        pltpu.VMEM((K, CHUNK), jnp.int32),
        pltpu.VMEM((CHUNK, 16), jnp.float32),
        pltpu.VMEM_SHARED((N_ACC, 16), jnp.float32),
    ],
    compiler_params=_sc_params,
)
def _deg_kernel(rows_hbm, zeros16_hbm, ones16_hbm, out_hbm, rowv, onesv, acc):
    c = lax.axis_index("c")
    s = lax.axis_index("s")
    wid = c * NS + s
    pltpu.sync_copy(rows_hbm.at[wid], rowv)
    pltpu.sync_copy(ones16_hbm, onesv)
    pltpu.sync_copy(zeros16_hbm.at[pl.ds(s * RPS, RPS)], acc.at[pl.ds(s * RPS, RPS)])
    plsc.subcore_barrier()

    @pl.loop(0, K)
    def _(k):
        pltpu.sync_copy(onesv, acc.at[rowv.at[k]], add=True)

    plsc.subcore_barrier()
    pltpu.sync_copy(acc.at[pl.ds(s * RPS, RPS)], out_hbm.at[c, pl.ds(s * RPS, RPS)])


@functools.partial(
    pl.kernel,
    out_type=jax.ShapeDtypeStruct((NC, N_ACC, CP), jnp.float32),
    mesh=_mesh,
    scratch_types=[
        pltpu.VMEM((K, CHUNK), jnp.int32),
        pltpu.VMEM((K, CHUNK), jnp.int32),
        pltpu.VMEM((NBUF, CHUNK, CP), jnp.float32),
        pltpu.VMEM_SHARED((N_ACC, CP), jnp.float32),
        pltpu.SemaphoreType.DMA((NBUF,)),
    ],
    compiler_params=_sc_params,
)
def _prop_kernel(z_hbm, zeros48_hbm, rows_hbm, cols_hbm, out_hbm,
                 rowv, colv, gbuf, acc, gsem):
    c = lax.axis_index("c")
    s = lax.axis_index("s")
    wid = c * NS + s
    pltpu.sync_copy(rows_hbm.at[wid], rowv)
    pltpu.sync_copy(cols_hbm.at[wid], colv)

    @pl.when(c == 0)
    def _():
        pltpu.sync_copy(z_hbm.at[pl.ds(s * RPS, RPS)], acc.at[pl.ds(s * RPS, RPS)])

    @pl.when(c == 1)
    def _():
        pltpu.sync_copy(zeros48_hbm.at[pl.ds(s * RPS, RPS)],
                        acc.at[pl.ds(s * RPS, RPS)])

    plsc.subcore_barrier()

    for b in range(NBUF):
        pltpu.async_copy(z_hbm.at[colv.at[b]], gbuf.at[b], gsem.at[b])

    @pl.loop(0, K // NBUF - 1)
    def _(g):
        for b in range(NBUF):
            k = g * NBUF + b
            pltpu.make_async_copy(z_hbm.at[colv.at[k]], gbuf.at[b],
                                  gsem.at[b]).wait()
            pltpu.sync_copy(gbuf.at[b], acc.at[rowv.at[k]], add=True)
            pltpu.async_copy(z_hbm.at[colv.at[k + NBUF]], gbuf.at[b],
                             gsem.at[b])

    for b in range(NBUF):
        k = K - NBUF + b
        pltpu.make_async_copy(z_hbm.at[colv.at[k]], gbuf.at[b], gsem.at[b]).wait()
        pltpu.sync_copy(gbuf.at[b], acc.at[rowv.at[k]], add=True)

    plsc.subcore_barrier()
    pltpu.sync_copy(acc.at[pl.ds(s * RPS, RPS)], out_hbm.at[c, pl.ds(s * RPS, RPS)])


def _mm_body(x_ref, w_ref, cnt_ref, z_ref, deg_ref):
    cnt = cnt_ref[...]
    deg = 1.0 + cnt[0, :, 0] + cnt[1, :, 0]
    y = jnp.dot(x_ref[...], w_ref[...], preferred_element_type=jnp.float32)
    z_ref[...] = y * lax.rsqrt(deg)[:, None]
    deg_ref[...] = deg[:, None]


def _tc_matmul(xp, wp, cnt):
    return pl.pallas_call(
        _mm_body,
        out_shape=(
            jax.ShapeDtypeStruct((N_ACC, CP), jnp.float32),
            jax.ShapeDtypeStruct((N_ACC, 1), jnp.float32),
        ),
    )(xp, wp, cnt)


def _mid_body(t_ref, deg_ref, z_ref):
    t = t_ref[...]
    z_ref[...] = (t[0] + t[1]) / deg_ref[...]


def _tc_mid(t, deg):
    return pl.pallas_call(
        _mid_body,
        out_shape=jax.ShapeDtypeStruct((N_ACC, CP), jnp.float32),
    )(t, deg)


def _fin_body(t_ref, deg_ref, b_ref, o_ref):
    t = t_ref[...]
    o_ref[...] = (t[0] + t[1]) * lax.rsqrt(deg_ref[...]) + b_ref[...]


def _tc_fin(t, deg, bp):
    return pl.pallas_call(
        _fin_body,
        out_shape=jax.ShapeDtypeStruct((N_ACC, CP), jnp.float32),
    )(t, deg, bp)


@jax.jit
def kernel(X, edge_index, W, b):
    ei = edge_index.astype(jnp.int32)
    pad = E_PAD - E
    padv = jax.lax.iota(jnp.int32, pad) % CHUNK
    rows = jnp.concatenate([ei[:, 0], N + padv])
    cols = jnp.concatenate([ei[:, 1], padv])
    rows = rows.reshape(NW, K, CHUNK)
    cols = cols.reshape(NW, K, CHUNK)

    xp = jnp.zeros((N_ACC, D), jnp.float32).at[:N].set(X)
    wp = jnp.zeros((D, CP), jnp.float32).at[:, :C].set(W.T)
    bp = jnp.zeros((1, CP), jnp.float32).at[0, :C].set(b)

    zeros16 = jnp.zeros((N_ACC, 16), jnp.float32)
    zeros48 = jnp.zeros((N_ACC, CP), jnp.float32)
    ones16 = jnp.ones((CHUNK, 16), jnp.float32)

    cnt = _deg_kernel(rows, zeros16, ones16)
    z0, deg = _tc_matmul(xp, wp, cnt)
    t1 = _prop_kernel(z0, zeros48, rows, cols)
    z1 = _tc_mid(t1, deg)
    t2 = _prop_kernel(z1, zeros48, rows, cols)
    out = _tc_fin(t2, deg, bp)
    return out[:N, :C]
